# Initial kernel scaffold; baseline (speedup 1.0000x reference)
import jax
import jax.numpy as jnp
from jax.experimental import pallas as pl


def kernel(x, values, skip_w, skip_b, grid):
    raise NotImplementedError("write your pallas kernel here")


# speedup vs baseline: 648.8474x; 648.8474x over previous
import jax
import jax.numpy as jnp
from jax.experimental import pallas as pl
from jax.experimental.pallas import tpu as pltpu

_K = 16


def _kan_body(grid_ref, x_ref, vt_ref, sw_ref, sb_ref, o_ref):
    xc = jnp.clip(x_ref[...], -1.0, 1.0)
    g0 = grid_ref[0]
    g_last = grid_ref[_K - 1]
    inv_h = (_K - 1) / (g_last - g0)
    acc = jax.lax.dot(xc, sw_ref[...], preferred_element_type=jnp.float32)
    acc = acc + sb_ref[...]
    for k in range(_K):
        gk = grid_ref[k]
        ck = jnp.maximum(1.0 - jnp.abs(xc - gk) * inv_h, 0.0)
        acc = acc + jax.lax.dot(ck, vt_ref[k], preferred_element_type=jnp.float32)
    o_ref[...] = acc


def kernel(x, values, skip_w, skip_b, grid):
    B, D = x.shape
    O = values.shape[0]
    vt = jnp.transpose(values, (2, 1, 0))
    sw = skip_w.T
    sb = skip_b.reshape(1, O)
    return pl.pallas_call(
        _kan_body,
        out_shape=jax.ShapeDtypeStruct((B, O), jnp.float32),
        in_specs=[
            pl.BlockSpec(memory_space=pltpu.SMEM),
            pl.BlockSpec(memory_space=pltpu.VMEM),
            pl.BlockSpec(memory_space=pltpu.VMEM),
            pl.BlockSpec(memory_space=pltpu.VMEM),
            pl.BlockSpec(memory_space=pltpu.VMEM),
        ],
        out_specs=pl.BlockSpec(memory_space=pltpu.VMEM),
    )(grid, x, vt, sw, sb)
